# baseline (speedup 1.0000x reference)
import functools

import jax
import jax.numpy as jnp
from jax import lax
from jax.experimental import pallas as pl
from jax.experimental.pallas import tpu as pltpu
from jax.experimental.pallas import tpu_sc as plsc

D_MODEL = 768
MAX_IDS = 50
NSLOTS = 64
N_TOKENS = 4 * 8192
N_WORKERS = 32
CHUNK = N_TOKENS // N_WORKERS
TILE = 2048


def _sc_presence_body(chrom_hbm, part_hbm, chrom_v, pres_v):
    c = lax.axis_index("c")
    s = lax.axis_index("s")
    wid = s * 2 + c
    zeros = jnp.zeros((16,), jnp.int32)
    ones = jnp.ones((16,), jnp.int32)
    pltpu.sync_copy(chrom_hbm.at[pl.ds(wid * CHUNK, CHUNK)], chrom_v)
    for k in range(NSLOTS // 16):
        pres_v[0, pl.ds(16 * k, 16)] = zeros
    for j in range(CHUNK // 16):
        vals = chrom_v[pl.ds(16 * j, 16)]
        plsc.store_scatter(pres_v, [zeros, vals], ones)
    pltpu.sync_copy(pres_v, part_hbm.at[pl.ds(wid, 1)])


@functools.cache
def _sc_presence():
    return pl.kernel(
        _sc_presence_body,
        out_type=jax.ShapeDtypeStruct((N_WORKERS, NSLOTS), jnp.int32),
        mesh=plsc.VectorSubcoreMesh(core_axis_name="c", subcore_axis_name="s"),
        scratch_types=[
            pltpu.VMEM((CHUNK,), jnp.int32),
            pltpu.VMEM((1, NSLOTS), jnp.int32),
        ],
        compiler_params=pltpu.CompilerParams(needs_layout_passes=False),
    )


def _dense_body(part_ref, pe_ref, chrom_ref, x_ref, out_ref, table_scr):
    @pl.when(pl.program_id(0) == 0)
    def _build_table():
        cnt = jnp.sum(part_ref[...], axis=0, keepdims=True)
        pres = (cnt > 0).astype(jnp.float32)
        tri = (lax.broadcasted_iota(jnp.int32, (NSLOTS, NSLOTS), 0)
               <= lax.broadcasted_iota(jnp.int32, (NSLOTS, NSLOTS), 1))
        incl = jnp.dot(pres, tri.astype(jnp.float32),
                       preferred_element_type=jnp.float32)
        rank = jnp.minimum((incl - pres).astype(jnp.int32), MAX_IDS - 1)
        rank_b = jnp.broadcast_to(rank, (NSLOTS, NSLOTS))
        row_iota = lax.broadcasted_iota(jnp.int32, (NSLOTS, NSLOTS), 0)
        perm = (rank_b == row_iota).astype(jnp.bfloat16)
        table = lax.dot_general(perm, pe_ref[...],
                                (((0,), (0,)), ((), ())),
                                preferred_element_type=jnp.float32)
        table_scr[...] = table.astype(jnp.bfloat16)

    cc = chrom_ref[...]
    iota = lax.broadcasted_iota(jnp.int32, (TILE, NSLOTS), 1)
    onehot = (cc == iota).astype(jnp.bfloat16)
    pe_rows = jnp.dot(onehot, table_scr[...],
                      preferred_element_type=jnp.float32)
    out_ref[...] = x_ref[...] + pe_rows


def _dense(partials, pe_pad, chrom_col, x2):
    grid = (N_TOKENS // TILE,)
    return pl.pallas_call(
        _dense_body,
        grid=grid,
        in_specs=[
            pl.BlockSpec((N_WORKERS, NSLOTS), lambda i: (0, 0)),
            pl.BlockSpec((NSLOTS, D_MODEL), lambda i: (0, 0)),
            pl.BlockSpec((TILE, 1), lambda i: (i, 0)),
            pl.BlockSpec((TILE, D_MODEL), lambda i: (i, 0)),
        ],
        out_specs=pl.BlockSpec((TILE, D_MODEL), lambda i: (i, 0)),
        out_shape=jax.ShapeDtypeStruct((N_TOKENS, D_MODEL), jnp.float32),
        scratch_shapes=[pltpu.VMEM((NSLOTS, D_MODEL), jnp.bfloat16)],
    )(partials, pe_pad, chrom_col, x2)


def kernel(x, chromosomes, pe):
    chrom_flat = chromosomes.reshape(-1)
    partials = _sc_presence()(chrom_flat)
    pe_pad = jnp.pad(pe.astype(jnp.bfloat16),
                     ((0, NSLOTS - MAX_IDS), (0, 0)))
    x2 = x.reshape(N_TOKENS, D_MODEL)
    out2 = _dense(partials, pe_pad, chrom_flat.reshape(N_TOKENS, 1), x2)
    return out2.reshape(x.shape)


# speedup vs baseline: 1.0063x; 1.0063x over previous
import functools

import jax
import jax.numpy as jnp
from jax import lax
from jax.experimental import pallas as pl
from jax.experimental.pallas import tpu as pltpu
from jax.experimental.pallas import tpu_sc as plsc

D_MODEL = 768
MAX_IDS = 50
NSLOTS = 64
N_TOKENS = 4 * 8192
N_WORKERS = 32
CHUNK = N_TOKENS // N_WORKERS
TILE = 4096


def _sc_presence_body(chrom_hbm, part_hbm, chrom_v, pres_v):
    c = lax.axis_index("c")
    s = lax.axis_index("s")
    wid = s * 2 + c
    zeros = jnp.zeros((16,), jnp.int32)
    ones = jnp.ones((16,), jnp.int32)
    pltpu.sync_copy(chrom_hbm.at[pl.ds(wid * CHUNK, CHUNK)], chrom_v)
    for k in range(NSLOTS // 16):
        pres_v[0, pl.ds(16 * k, 16)] = zeros
    for j in range(CHUNK // 16):
        vals = chrom_v[pl.ds(16 * j, 16)]
        plsc.store_scatter(pres_v, [zeros, vals], ones)
    pltpu.sync_copy(pres_v, part_hbm.at[pl.ds(wid, 1)])


@functools.cache
def _sc_presence():
    return pl.kernel(
        _sc_presence_body,
        out_type=jax.ShapeDtypeStruct((N_WORKERS, NSLOTS), jnp.int32),
        mesh=plsc.VectorSubcoreMesh(core_axis_name="c", subcore_axis_name="s"),
        scratch_types=[
            pltpu.VMEM((CHUNK,), jnp.int32),
            pltpu.VMEM((1, NSLOTS), jnp.int32),
        ],
        compiler_params=pltpu.CompilerParams(needs_layout_passes=False),
    )


def _dense_body(part_ref, pe_ref, chrom_ref, x_ref, out_ref, table_scr):
    @pl.when(pl.program_id(0) == 0)
    def _build_table():
        cnt = jnp.sum(part_ref[...], axis=0, keepdims=True)
        pres = (cnt > 0).astype(jnp.float32)
        tri = (lax.broadcasted_iota(jnp.int32, (NSLOTS, NSLOTS), 0)
               <= lax.broadcasted_iota(jnp.int32, (NSLOTS, NSLOTS), 1))
        incl = jnp.dot(pres, tri.astype(jnp.float32),
                       preferred_element_type=jnp.float32)
        rank = jnp.minimum((incl - pres).astype(jnp.int32), MAX_IDS - 1)
        rank_b = jnp.broadcast_to(rank, (NSLOTS, NSLOTS))
        row_iota = lax.broadcasted_iota(jnp.int32, (NSLOTS, NSLOTS), 0)
        perm = (rank_b == row_iota).astype(jnp.bfloat16)
        table = lax.dot_general(perm, pe_ref[...],
                                (((0,), (0,)), ((), ())),
                                preferred_element_type=jnp.float32)
        table_scr[...] = table.astype(jnp.bfloat16)

    cc = chrom_ref[...]
    iota = lax.broadcasted_iota(jnp.int32, (TILE, NSLOTS), 1)
    onehot = (cc == iota).astype(jnp.bfloat16)
    pe_rows = jnp.dot(onehot, table_scr[...],
                      preferred_element_type=jnp.float32)
    out_ref[...] = x_ref[...] + pe_rows


def _dense(partials, pe_pad, chrom_col, x2):
    grid = (N_TOKENS // TILE,)
    return pl.pallas_call(
        _dense_body,
        grid=grid,
        in_specs=[
            pl.BlockSpec((N_WORKERS, NSLOTS), lambda i: (0, 0)),
            pl.BlockSpec((NSLOTS, D_MODEL), lambda i: (0, 0)),
            pl.BlockSpec((TILE, 1), lambda i: (i, 0)),
            pl.BlockSpec((TILE, D_MODEL), lambda i: (i, 0)),
        ],
        out_specs=pl.BlockSpec((TILE, D_MODEL), lambda i: (i, 0)),
        out_shape=jax.ShapeDtypeStruct((N_TOKENS, D_MODEL), jnp.float32),
        scratch_shapes=[pltpu.VMEM((NSLOTS, D_MODEL), jnp.bfloat16)],
    )(partials, pe_pad, chrom_col, x2)


def kernel(x, chromosomes, pe):
    chrom_flat = chromosomes.reshape(-1)
    partials = _sc_presence()(chrom_flat)
    pe_pad = jnp.pad(pe.astype(jnp.bfloat16),
                     ((0, NSLOTS - MAX_IDS), (0, 0)))
    x2 = x.reshape(N_TOKENS, D_MODEL)
    out2 = _dense(partials, pe_pad, chrom_flat.reshape(N_TOKENS, 1), x2)
    return out2.reshape(x.shape)
